# baseline (speedup 1.0000x reference)
import functools

import jax
import jax.numpy as jnp
from jax import lax
from jax.experimental import pallas as pl
from jax.experimental.pallas import tpu as pltpu
from jax.experimental.pallas import tpu_sc as plsc

_N = 10000
_E = 160000
_HID = 256
_NW = 32
_PT = 320
_N2 = _NW * _PT
_CH = 4000
_NCH = _E // _CH
_SB = 64
_SLOT = _CH + _SB
_NEG = -3.0e38
_THRESH = -1.0e37
_FB = _HID // 16
_BLK = 512
_GRID = _N2 // _BLK

_mesh = plsc.VectorSubcoreMesh(core_axis_name="c", subcore_axis_name="s")
_sc_params = pltpu.CompilerParams(needs_layout_passes=False)


def _tile_id():
    return lax.axis_index("s") * 2 + lax.axis_index("c")


@functools.partial(
    pl.kernel,
    out_type=[
        jax.ShapeDtypeStruct((_NW, _NCH, _SLOT), jnp.int32),
        jax.ShapeDtypeStruct((_NW, _NCH, _SLOT), jnp.int32),
        jax.ShapeDtypeStruct((_NW, _NCH, 16), jnp.int32),
        jax.ShapeDtypeStruct((_N2,), jnp.float32),
    ],
    mesh=_mesh,
    scratch_types=[
        pltpu.VMEM((_CH,), jnp.int32),
        pltpu.VMEM((_CH,), jnp.int32),
        pltpu.VMEM((_CH,), jnp.float32),
        pltpu.VMEM((_SLOT,), jnp.int32),
        pltpu.VMEM((_SLOT,), jnp.int32),
        pltpu.VMEM((16,), jnp.int32),
        pltpu.VMEM((_PT,), jnp.float32),
    ],
    compiler_params=_sc_params,
)
def _prep(src_h, dst_h, mask_h, srcs_h, ldst_h, cnts_h, hm_h,
          sbuf, dbuf, mbuf, csrc, cldst, cntbuf, hmbuf):
    w = _tile_id()
    d0 = w * _PT
    zero16 = jnp.zeros((16,), jnp.float32)
    one16 = jnp.ones((16,), jnp.float32)
    lane = lax.iota(jnp.int32, 16)
    padldst = jnp.full((16,), _PT, jnp.int32)

    def initr(r, _):
        hmbuf[pl.ds(r * 16, 16)] = zero16
        return 0
    lax.fori_loop(0, _PT // 16, initr, 0)

    def chunk(ch, _):
        pltpu.sync_copy(src_h.at[pl.ds(ch * _CH, _CH)], sbuf)
        pltpu.sync_copy(dst_h.at[pl.ds(ch * _CH, _CH)], dbuf)
        pltpu.sync_copy(mask_h.at[pl.ds(ch * _CH, _CH)], mbuf)

        def inner(i, cnt):
            dv = dbuf[pl.ds(i * 16, 16)]
            sv = sbuf[pl.ds(i * 16, 16)]
            mv = mbuf[pl.ds(i * 16, 16)]
            inr = (dv >= d0) & (dv < d0 + _PT)
            keep = inr & (mv > 0.5)
            msk = inr & (mv < 0.5)
            ldv = dv - d0
            pref = plsc.cumsum(jnp.where(keep, 1, 0).astype(jnp.int32))
            pos = cnt + pref - 1
            plsc.store_scatter(csrc, [pos], sv, mask=keep)
            plsc.store_scatter(cldst, [pos], ldv, mask=keep)
            plsc.store_scatter(hmbuf, [ldv], one16, mask=msk)
            return cnt + pref[15]

        cnt = lax.fori_loop(0, _CH // 16, inner, jnp.int32(0))
        for q in range(_SB // 16):
            csrc[pl.ds(cnt + q * 16, 16)] = w * _PT + q * 16 + lane
            cldst[pl.ds(cnt + q * 16, 16)] = padldst
        cntbuf[...] = jnp.broadcast_to(cnt, (16,)).astype(jnp.int32)
        pltpu.sync_copy(cntbuf, cnts_h.at[w, ch])
        pltpu.sync_copy(csrc, srcs_h.at[w, ch])
        pltpu.sync_copy(cldst, ldst_h.at[w, ch])
        return 0

    lax.fori_loop(0, _NCH, chunk, 0)
    pltpu.sync_copy(hmbuf, hm_h.at[pl.ds(d0, _PT)])


@functools.partial(
    pl.kernel,
    out_type=jax.ShapeDtypeStruct((_N2, _HID), jnp.float32),
    mesh=_mesh,
    scratch_types=[
        pltpu.VMEM((_PT + 1, _HID), jnp.float32),
        pltpu.VMEM((_SB, _HID), jnp.float32),
        pltpu.VMEM((_SLOT,), jnp.int32),
        pltpu.VMEM((_SLOT,), jnp.int32),
        pltpu.VMEM((_NCH, 16), jnp.int32),
        pltpu.SemaphoreType.DMA,
        pltpu.SemaphoreType.DMA,
        pltpu.SemaphoreType.DMA,
        pltpu.SemaphoreType.DMA,
    ],
    compiler_params=_sc_params,
)
def _seg(a_h, srcs_h, ldst_h, cnts_h, amax_h, acc, rows, sidx, lbuf, cbuf,
         sem0, sem1, sem2, sem3):
    sems = [sem0, sem1, sem2, sem3]
    w = _tile_id()
    d0 = w * _PT
    negv = jnp.full((16,), _NEG, jnp.float32)

    def initr(r, _):
        for f in range(_FB):
            acc[r, pl.ds(f * 16, 16)] = negv
        return 0
    lax.fori_loop(0, _PT + 1, initr, 0)

    pltpu.sync_copy(cnts_h.at[w], cbuf)

    def chunk(ch, _):
        cnt = cbuf[ch][0]
        nsub = (cnt + (_SB - 1)) // _SB

        @pl.when(nsub > 0)
        def _():
            pltpu.sync_copy(srcs_h.at[w, ch], sidx)
            pltpu.sync_copy(ldst_h.at[w, ch], lbuf)

            def sub(sb, _):
                pltpu.async_copy(
                    a_h.at[sidx.at[pl.ds(sb * _SB, _SB)]], rows, sems[0]
                ).wait()

                def grp(g, _):
                    ldv = lbuf[pl.ds(sb * _SB + g * 16, 16)]
                    for k in range(16):
                        ld = ldv[k]
                        j = g * 16 + k
                        avs = [acc[ld, pl.ds(f * 16, 16)] for f in range(_FB)]
                        rvs = [rows[j, pl.ds(f * 16, 16)] for f in range(_FB)]
                        mx = [jnp.maximum(a, r) for a, r in zip(avs, rvs)]
                        for f in range(_FB):
                            acc[ld, pl.ds(f * 16, 16)] = mx[f]
                    return 0

                lax.fori_loop(0, _SB // 16, grp, 0)
                return 0

            lax.fori_loop(0, nsub, sub, 0)
        return 0

    lax.fori_loop(0, _NCH, chunk, 0)
    pltpu.sync_copy(acc.at[pl.ds(0, _PT)], amax_h.at[pl.ds(d0, _PT)])


def _full(shape):
    return pl.BlockSpec(shape, lambda i: (0, 0))


def _tc1_body(x_ref, wt_ref, wp_ref, bt_ref, bp_ref, a_ref, c_ref):
    x = x_ref[...]
    wt = wt_ref[...]
    wp = wp_ref[...]
    a = x[:, 0:1] * wt[0:1, :] + x[:, 1:2] * wt[1:2, :]
    wd = wp - wt
    c = x[:, 0:1] * wd[0:1, :] + x[:, 1:2] * wd[1:2, :]
    a_ref[...] = a
    c_ref[...] = c + bt_ref[...] + bp_ref[...]


_tc1 = pl.pallas_call(
    _tc1_body,
    grid=(_GRID,),
    in_specs=[
        pl.BlockSpec((_BLK, 2), lambda i: (i, 0)),
        _full((2, _HID)), _full((2, _HID)),
        _full((1, _HID)), _full((1, _HID)),
    ],
    out_specs=[
        pl.BlockSpec((_BLK, _HID), lambda i: (i, 0)),
        pl.BlockSpec((_BLK, _HID), lambda i: (i, 0)),
    ],
    out_shape=[
        jax.ShapeDtypeStruct((_N2, _HID), jnp.float32),
        jax.ShapeDtypeStruct((_N2, _HID), jnp.float32),
    ],
)


def _combine(amax, cprev, hmv):
    u = amax + cprev
    return jnp.where(
        amax > _THRESH,
        jnp.where(hmv > 0.5, jnp.maximum(u, 0.0), u),
        0.0,
    )


def _tcl_body(amax_ref, cprev_ref, hm_ref, wt_ref, wp_ref, bt_ref, bp_ref,
              a_ref, c_ref):
    h = _combine(amax_ref[...], cprev_ref[...], hm_ref[...])
    wt = wt_ref[...]
    a_ref[...] = lax.dot(h, wt, preferred_element_type=jnp.float32)
    c_ref[...] = (
        lax.dot(h, wp_ref[...] - wt, preferred_element_type=jnp.float32)
        + bt_ref[...] + bp_ref[...]
    )


_tcl = pl.pallas_call(
    _tcl_body,
    grid=(_GRID,),
    in_specs=[
        pl.BlockSpec((_BLK, _HID), lambda i: (i, 0)),
        pl.BlockSpec((_BLK, _HID), lambda i: (i, 0)),
        pl.BlockSpec((_BLK, 1), lambda i: (i, 0)),
        _full((_HID, _HID)), _full((_HID, _HID)),
        _full((1, _HID)), _full((1, _HID)),
    ],
    out_specs=[
        pl.BlockSpec((_BLK, _HID), lambda i: (i, 0)),
        pl.BlockSpec((_BLK, _HID), lambda i: (i, 0)),
    ],
    out_shape=[
        jax.ShapeDtypeStruct((_N2, _HID), jnp.float32),
        jax.ShapeDtypeStruct((_N2, _HID), jnp.float32),
    ],
)


def _head_body(amax_ref, cprev_ref, hm_ref, wl_ref, bl_ref, wl2_ref, bl2_ref,
               wo_ref, bo_ref, out_ref):
    i = pl.program_id(0)
    h = _combine(amax_ref[...], cprev_ref[...], hm_ref[...])
    t = jnp.maximum(
        lax.dot(h, wl_ref[...], preferred_element_type=jnp.float32)
        + bl_ref[...], 0.0)
    t2 = jnp.maximum(
        lax.dot(t, wl2_ref[...], preferred_element_type=jnp.float32)
        + bl2_ref[...], 0.0)
    o = (lax.dot(t2, wo_ref[...], preferred_element_type=jnp.float32)
         + bo_ref[...])
    rid = i * _BLK + lax.broadcasted_iota(jnp.int32, (_BLK, 1), 0)
    ps = jnp.sum(jnp.where(rid < _N, o, 0.0))

    @pl.when(i == 0)
    def _():
        out_ref[...] = jnp.zeros_like(out_ref)

    out_ref[...] += ps

    @pl.when(i == _GRID - 1)
    def _():
        out_ref[...] *= (1.0 / _N)


_head = pl.pallas_call(
    _head_body,
    grid=(_GRID,),
    in_specs=[
        pl.BlockSpec((_BLK, _HID), lambda i: (i, 0)),
        pl.BlockSpec((_BLK, _HID), lambda i: (i, 0)),
        pl.BlockSpec((_BLK, 1), lambda i: (i, 0)),
        _full((_HID, 400)), _full((1, 400)),
        _full((400, 200)), _full((1, 200)),
        _full((200, 1)), _full((1, 1)),
    ],
    out_specs=pl.BlockSpec((1, 1), lambda i: (0, 0)),
    out_shape=jax.ShapeDtypeStruct((1, 1), jnp.float32),
)


def kernel(coords, edge_index, mask,
           Wt1, bt1, Wp1, bp1,
           Wt2, bt2, Wp2, bp2,
           Wt3, bt3, Wp3, bp3,
           Wl, bl, Wl2, bl2, Wo, bo):
    src = edge_index[0].astype(jnp.int32)
    dst = edge_index[1].astype(jnp.int32)
    srcs, ldsts, cnts, hm = _prep(src, dst, mask)
    hm2 = hm.reshape(_N2, 1)
    cpad = jnp.concatenate(
        [coords, jnp.zeros((_N2 - _N, 2), coords.dtype)], axis=0)

    a1, c1 = _tc1(cpad, Wt1, Wp1, bt1.reshape(1, -1), bp1.reshape(1, -1))
    amax1 = _seg(a1, srcs, ldsts, cnts)
    a2, c2 = _tcl(amax1, c1, hm2, Wt2, Wp2,
                  bt2.reshape(1, -1), bp2.reshape(1, -1))
    amax2 = _seg(a2, srcs, ldsts, cnts)
    a3, c3 = _tcl(amax2, c2, hm2, Wt3, Wp3,
                  bt3.reshape(1, -1), bp3.reshape(1, -1))
    amax3 = _seg(a3, srcs, ldsts, cnts)
    out = _head(amax3, c3, hm2, Wl, bl.reshape(1, -1),
                Wl2, bl2.reshape(1, -1), Wo, bo.reshape(1, -1))
    return out.reshape(-1)


# speedup vs baseline: 2.2229x; 2.2229x over previous
import functools

import jax
import jax.numpy as jnp
from jax import lax
from jax.experimental import pallas as pl
from jax.experimental.pallas import tpu as pltpu
from jax.experimental.pallas import tpu_sc as plsc

_N = 10000
_E = 160000
_HID = 256
_NW = 32
_PT = 320
_N2 = _NW * _PT
_CH = 4000
_NCH = _E // _CH
_SB = 64
_SLOT = _CH + _SB
_PC = 4096
_LCAP = _E + 2 * _PC
_NEG = -3.0e38
_THRESH = -1.0e37
_FB = _HID // 16
_BLK = 512
_GRID = _N2 // _BLK

_mesh = plsc.VectorSubcoreMesh(core_axis_name="c", subcore_axis_name="s")
_sc_params = pltpu.CompilerParams(needs_layout_passes=False)


def _tile_id():
    return lax.axis_index("s") * 2 + lax.axis_index("c")


@functools.partial(
    pl.kernel,
    out_type=[
        jax.ShapeDtypeStruct((_NW * _LCAP,), jnp.int32),
        jax.ShapeDtypeStruct((_NW * _LCAP,), jnp.int32),
        jax.ShapeDtypeStruct((_NW * 16,), jnp.int32),
        jax.ShapeDtypeStruct((_N2,), jnp.float32),
    ],
    mesh=_mesh,
    scratch_types=[
        pltpu.VMEM((_CH,), jnp.int32),
        pltpu.VMEM((_CH,), jnp.int32),
        pltpu.VMEM((_CH,), jnp.int32),
        pltpu.VMEM((_CH,), jnp.int32),
        pltpu.VMEM((_CH,), jnp.float32),
        pltpu.VMEM((_CH,), jnp.float32),
        pltpu.VMEM((_SLOT,), jnp.int32),
        pltpu.VMEM((_SLOT,), jnp.int32),
        pltpu.VMEM((_SLOT,), jnp.int32),
        pltpu.VMEM((_SLOT,), jnp.int32),
        pltpu.VMEM((16,), jnp.int32),
        pltpu.VMEM((_PT,), jnp.float32),
        pltpu.SemaphoreType.DMA,
        pltpu.SemaphoreType.DMA,
        pltpu.SemaphoreType.DMA,
    ],
    compiler_params=_sc_params,
)
def _prep(src_h, dst_h, mask_h, lsrc_h, lldst_h, tot_h, hm_h,
          sbuf0, sbuf1, dbuf0, dbuf1, mbuf0, mbuf1,
          csrc0, csrc1, cldst0, cldst1, totbuf, hmbuf, semA, semB, semC):
    w = _tile_id()
    d0 = w * _PT
    zero16 = jnp.zeros((16,), jnp.float32)
    one16 = jnp.ones((16,), jnp.float32)
    lane = lax.iota(jnp.int32, 16)

    def initr(r, _):
        hmbuf[pl.ds(r * 16, 16)] = zero16
        return 0
    lax.fori_loop(0, _PT // 16, initr, 0)

    def _fire_in(ch, sb_, db_, mb_, sem):
        pltpu.async_copy(src_h.at[pl.ds(ch * _CH, _CH)], sb_, sem)
        pltpu.async_copy(dst_h.at[pl.ds(ch * _CH, _CH)], db_, sem)
        pltpu.async_copy(mask_h.at[pl.ds(ch * _CH, _CH)], mb_, sem)

    def _wait_in(ch, sb_, db_, mb_, sem):
        pltpu.make_async_copy(src_h.at[pl.ds(ch * _CH, _CH)], sb_, sem).wait()
        pltpu.make_async_copy(dst_h.at[pl.ds(ch * _CH, _CH)], db_, sem).wait()
        pltpu.make_async_copy(mask_h.at[pl.ds(ch * _CH, _CH)], mb_, sem).wait()

    _fire_in(0, sbuf0, dbuf0, mbuf0, semA)

    def chunk(ch, off):
        bs = ch % 2

        @pl.when(ch >= 2)
        def _():
            pltpu.make_async_copy(
                csrc0, lsrc_h.at[pl.ds(pl.multiple_of(w * _LCAP, 8), _SLOT)],
                semC).wait()
            pltpu.make_async_copy(
                csrc0, lsrc_h.at[pl.ds(pl.multiple_of(w * _LCAP, 8), _SLOT)],
                semC).wait()

        def do(sb_, db_, mb_, cs_, cl_, semIn, nsb_, ndb_, nmb_, semN):
            @pl.when(ch + 1 < _NCH)
            def _():
                _fire_in(ch + 1, nsb_, ndb_, nmb_, semN)

            _wait_in(ch, sb_, db_, mb_, semIn)

            def inner(i, cnt):
                dv = db_[pl.ds(i * 16, 16)]
                sv = sb_[pl.ds(i * 16, 16)]
                mv = mb_[pl.ds(i * 16, 16)]
                inr = (dv >= d0) & (dv < d0 + _PT)
                keep = inr & (mv > 0.5)
                msk = inr & (mv < 0.5)
                ldv = dv - d0
                pref = plsc.cumsum(jnp.where(keep, 1, 0).astype(jnp.int32))
                pos = cnt + pref - 1
                plsc.store_scatter(cs_, [pos], sv, mask=keep)
                plsc.store_scatter(cl_, [pos], ldv, mask=keep)
                plsc.store_scatter(hmbuf, [ldv], one16, mask=msk)
                return cnt + pref[15]

            cnt = lax.fori_loop(0, _CH // 16, inner, jnp.int32(0))

            @pl.when(cnt > 0)
            def _():
                lastS = cs_[pl.ds(cnt - 1, 16)][0]
                lastL = cl_[pl.ds(cnt - 1, 16)][0]
                cs_[pl.ds(cnt, 16)] = jnp.broadcast_to(lastS, (16,)).astype(jnp.int32)
                cl_[pl.ds(cnt, 16)] = jnp.broadcast_to(lastL, (16,)).astype(jnp.int32)

            pltpu.async_copy(
                cs_, lsrc_h.at[pl.ds(pl.multiple_of(w * _LCAP + off, 8), _SLOT)],
                semC)
            pltpu.async_copy(
                cl_, lldst_h.at[pl.ds(pl.multiple_of(w * _LCAP + off, 8), _SLOT)],
                semC)
            return cnt

        cnt = lax.cond(
            bs == 0,
            lambda: do(sbuf0, dbuf0, mbuf0, csrc0, cldst0, semA,
                       sbuf1, dbuf1, mbuf1, semB),
            lambda: do(sbuf1, dbuf1, mbuf1, csrc1, cldst1, semB,
                       sbuf0, dbuf0, mbuf0, semA),
        )
        cnt8 = jnp.where(cnt > 0, (cnt + 7) & ~7, 0)
        return off + cnt8

    total = lax.fori_loop(0, _NCH, chunk, jnp.int32(0))

    for _i in range(4):
        pltpu.make_async_copy(
            csrc0, lsrc_h.at[pl.ds(pl.multiple_of(w * _LCAP, 8), _SLOT)],
            semC).wait()

    for q in range(_SB // 16):
        csrc0[pl.ds(q * 16, 16)] = d0 + q * 16 + lane
        cldst0[pl.ds(q * 16, 16)] = jnp.full((16,), _PT, jnp.int32)
    pltpu.sync_copy(csrc0.at[pl.ds(0, _SB)],
                    lsrc_h.at[pl.ds(pl.multiple_of(w * _LCAP + total, 8), _SB)])
    pltpu.sync_copy(cldst0.at[pl.ds(0, _SB)],
                    lldst_h.at[pl.ds(pl.multiple_of(w * _LCAP + total, 8), _SB)])

    totbuf[...] = jnp.broadcast_to(total, (16,)).astype(jnp.int32)
    pltpu.sync_copy(totbuf, tot_h.at[pl.ds(pl.multiple_of(w * 16, 8), 16)])
    pltpu.sync_copy(hmbuf, hm_h.at[pl.ds(d0, _PT)])


@functools.partial(
    pl.kernel,
    out_type=jax.ShapeDtypeStruct((_N2, _HID), jnp.float32),
    mesh=_mesh,
    scratch_types=[
        pltpu.VMEM((_PT + 1, _HID), jnp.float32),
        pltpu.VMEM((_SB, _HID), jnp.float32),
        pltpu.VMEM((_SB, _HID), jnp.float32),
        pltpu.VMEM((_PC,), jnp.int32),
        pltpu.VMEM((_PC,), jnp.int32),
        pltpu.VMEM((16,), jnp.int32),
        pltpu.SemaphoreType.DMA,
        pltpu.SemaphoreType.DMA,
    ],
    compiler_params=_sc_params,
)
def _seg(a_h, lsrc_h, lldst_h, tot_h, amax_h,
         acc, rows0, rows1, sidx, lbuf, tb, sem0, sem1):
    w = _tile_id()
    d0 = w * _PT
    negv = jnp.full((16,), _NEG, jnp.float32)

    def initr(r, _):
        for f in range(_FB):
            acc[r, pl.ds(f * 16, 16)] = negv
        return 0
    lax.fori_loop(0, _PT + 1, initr, 0)

    pltpu.sync_copy(tot_h.at[pl.ds(pl.multiple_of(w * 16, 8), 16)], tb)
    total = tb[...][0]
    npc = (total + _PC - 1) // _PC

    def _fire(b, rows_, sem):
        pltpu.async_copy(a_h.at[sidx.at[pl.ds(b * _SB, _SB)]], rows_, sem)

    def _wait(b, rows_, sem):
        pltpu.make_async_copy(
            a_h.at[sidx.at[pl.ds(b * _SB, _SB)]], rows_, sem).wait()

    def piece(p, _):
        base = pl.multiple_of(p * _PC, 8)
        rem = total - base
        nb = (jnp.minimum(rem, _PC) + _SB - 1) // _SB
        pltpu.sync_copy(
            lsrc_h.at[pl.ds(pl.multiple_of(w * _LCAP + base, 8), _PC)], sidx)
        pltpu.sync_copy(
            lldst_h.at[pl.ds(pl.multiple_of(w * _LCAP + base, 8), _PC)], lbuf)
        _fire(0, rows0, sem0)

        def bat(b, _):
            bs = b % 2

            def do(rows_, sem_, nrows_, nsem_):
                @pl.when(b + 1 < nb)
                def _():
                    _fire(b + 1, nrows_, nsem_)

                _wait(b, rows_, sem_)

                def grp(g, _):
                    ldv = lbuf[pl.ds(b * _SB + g * 16, 16)]
                    for k in range(16):
                        ld = ldv[k]
                        j = g * 16 + k
                        avs = [acc[ld, pl.ds(f * 16, 16)] for f in range(_FB)]
                        rvs = [rows_[j, pl.ds(f * 16, 16)] for f in range(_FB)]
                        mx = [jnp.maximum(a, r) for a, r in zip(avs, rvs)]
                        for f in range(_FB):
                            acc[ld, pl.ds(f * 16, 16)] = mx[f]
                    return 0

                lax.fori_loop(0, _SB // 16, grp, 0)
                return 0

            lax.cond(
                bs == 0,
                lambda: do(rows0, sem0, rows1, sem1),
                lambda: do(rows1, sem1, rows0, sem0),
            )
            return 0

        lax.fori_loop(0, nb, bat, 0)
        return 0

    lax.fori_loop(0, npc, piece, 0)
    pltpu.sync_copy(acc.at[pl.ds(0, _PT)], amax_h.at[pl.ds(d0, _PT)])


def _full(shape):
    return pl.BlockSpec(shape, lambda i: (0, 0))


def _tc1_body(x_ref, wt_ref, wp_ref, bt_ref, bp_ref, a_ref, c_ref):
    x = x_ref[...]
    wt = wt_ref[...]
    wp = wp_ref[...]
    a = x[:, 0:1] * wt[0:1, :] + x[:, 1:2] * wt[1:2, :]
    wd = wp - wt
    c = x[:, 0:1] * wd[0:1, :] + x[:, 1:2] * wd[1:2, :]
    a_ref[...] = a
    c_ref[...] = c + bt_ref[...] + bp_ref[...]


_tc1 = pl.pallas_call(
    _tc1_body,
    grid=(_GRID,),
    in_specs=[
        pl.BlockSpec((_BLK, 2), lambda i: (i, 0)),
        _full((2, _HID)), _full((2, _HID)),
        _full((1, _HID)), _full((1, _HID)),
    ],
    out_specs=[
        pl.BlockSpec((_BLK, _HID), lambda i: (i, 0)),
        pl.BlockSpec((_BLK, _HID), lambda i: (i, 0)),
    ],
    out_shape=[
        jax.ShapeDtypeStruct((_N2, _HID), jnp.float32),
        jax.ShapeDtypeStruct((_N2, _HID), jnp.float32),
    ],
)


def _combine(amax, cprev, hmv):
    u = amax + cprev
    return jnp.where(
        amax > _THRESH,
        jnp.where(hmv > 0.5, jnp.maximum(u, 0.0), u),
        0.0,
    )


def _tcl_body(amax_ref, cprev_ref, hm_ref, wt_ref, wp_ref, bt_ref, bp_ref,
              a_ref, c_ref):
    h = _combine(amax_ref[...], cprev_ref[...], hm_ref[...])
    wt = wt_ref[...]
    a_ref[...] = lax.dot(h, wt, preferred_element_type=jnp.float32)
    c_ref[...] = (
        lax.dot(h, wp_ref[...] - wt, preferred_element_type=jnp.float32)
        + bt_ref[...] + bp_ref[...]
    )


_tcl = pl.pallas_call(
    _tcl_body,
    grid=(_GRID,),
    in_specs=[
        pl.BlockSpec((_BLK, _HID), lambda i: (i, 0)),
        pl.BlockSpec((_BLK, _HID), lambda i: (i, 0)),
        pl.BlockSpec((_BLK, 1), lambda i: (i, 0)),
        _full((_HID, _HID)), _full((_HID, _HID)),
        _full((1, _HID)), _full((1, _HID)),
    ],
    out_specs=[
        pl.BlockSpec((_BLK, _HID), lambda i: (i, 0)),
        pl.BlockSpec((_BLK, _HID), lambda i: (i, 0)),
    ],
    out_shape=[
        jax.ShapeDtypeStruct((_N2, _HID), jnp.float32),
        jax.ShapeDtypeStruct((_N2, _HID), jnp.float32),
    ],
)


def _head_body(amax_ref, cprev_ref, hm_ref, wl_ref, bl_ref, wl2_ref, bl2_ref,
               wo_ref, bo_ref, out_ref):
    i = pl.program_id(0)
    h = _combine(amax_ref[...], cprev_ref[...], hm_ref[...])
    t = jnp.maximum(
        lax.dot(h, wl_ref[...], preferred_element_type=jnp.float32)
        + bl_ref[...], 0.0)
    t2 = jnp.maximum(
        lax.dot(t, wl2_ref[...], preferred_element_type=jnp.float32)
        + bl2_ref[...], 0.0)
    o = (lax.dot(t2, wo_ref[...], preferred_element_type=jnp.float32)
         + bo_ref[...])
    rid = i * _BLK + lax.broadcasted_iota(jnp.int32, (_BLK, 1), 0)
    ps = jnp.sum(jnp.where(rid < _N, o, 0.0))

    @pl.when(i == 0)
    def _():
        out_ref[...] = jnp.zeros_like(out_ref)

    out_ref[...] += ps

    @pl.when(i == _GRID - 1)
    def _():
        out_ref[...] *= (1.0 / _N)


_head = pl.pallas_call(
    _head_body,
    grid=(_GRID,),
    in_specs=[
        pl.BlockSpec((_BLK, _HID), lambda i: (i, 0)),
        pl.BlockSpec((_BLK, _HID), lambda i: (i, 0)),
        pl.BlockSpec((_BLK, 1), lambda i: (i, 0)),
        _full((_HID, 400)), _full((1, 400)),
        _full((400, 200)), _full((1, 200)),
        _full((200, 1)), _full((1, 1)),
    ],
    out_specs=pl.BlockSpec((1, 1), lambda i: (0, 0)),
    out_shape=jax.ShapeDtypeStruct((1, 1), jnp.float32),
)


def kernel(coords, edge_index, mask,
           Wt1, bt1, Wp1, bp1,
           Wt2, bt2, Wp2, bp2,
           Wt3, bt3, Wp3, bp3,
           Wl, bl, Wl2, bl2, Wo, bo):
    src = edge_index[0].astype(jnp.int32)
    dst = edge_index[1].astype(jnp.int32)
    lsrc, lldst, tot, hm = _prep(src, dst, mask)
    hm2 = hm.reshape(_N2, 1)
    cpad = jnp.concatenate(
        [coords, jnp.zeros((_N2 - _N, 2), coords.dtype)], axis=0)

    a1, c1 = _tc1(cpad, Wt1, Wp1, bt1.reshape(1, -1), bp1.reshape(1, -1))
    amax1 = _seg(a1, lsrc, lldst, tot)
    a2, c2 = _tcl(amax1, c1, hm2, Wt2, Wp2,
                  bt2.reshape(1, -1), bp2.reshape(1, -1))
    amax2 = _seg(a2, lsrc, lldst, tot)
    a3, c3 = _tcl(amax2, c2, hm2, Wt3, Wp3,
                  bt3.reshape(1, -1), bp3.reshape(1, -1))
    amax3 = _seg(a3, lsrc, lldst, tot)
    out = _head(amax3, c3, hm2, Wl, bl.reshape(1, -1),
                Wl2, bl2.reshape(1, -1), Wo, bo.reshape(1, -1))
    return out.reshape(-1)
